# baseline (speedup 1.0000x reference)
import jax
import jax.numpy as jnp
from jax import lax
from jax.experimental import pallas as pl
from jax.experimental.pallas import tpu as pltpu
from jax.experimental.pallas import tpu_sc as plsc

_T, _B, _C = 1024, 128, 192
_REMAIN = _T - (_T * 3) // 4
_NC, _NS = 2, 16
_NW = _NC * _NS
_SLABS_PER_W = _T // _NW
_OSLABS_PER_W = _REMAIN // _NW
_NBUF = 4
_L = 16
_SP = _B + 1


def _tr_body(src_hbm, tbl_hbm, in_bufs, tr_bufs, sem_i, sem_o):
    wid = lax.axis_index("s") * _NC + lax.axis_index("c")
    s0 = wid * _SLABS_PER_W
    rows16 = [lax.iota(jnp.int32, _L) + m * _L for m in range(_C // _L)]

    def in_cp(j, k):
        return pltpu.make_async_copy(
            src_hbm.at[pl.ds((s0 + j) * _C, _C)],
            in_bufs[k].at[:, pl.ds(0, _B)], sem_i[k])

    def out_cp(j, k):
        return pltpu.make_async_copy(
            tr_bufs[k], tbl_hbm.at[pl.ds((s0 + j) * _B, _B)], sem_o[k])

    def transpose(k):
        @pl.loop(0, _B, unroll=8)
        def _col(b):
            col = jnp.full((_L,), b, dtype=jnp.int32)
            for m in range(_C // _L):
                v = plsc.load_gather(in_bufs[k], [rows16[m], col])
                tr_bufs[k][b, pl.ds(m * _L, _L)] = v

    in_cp(0, 0).start()

    @pl.loop(0, _SLABS_PER_W, step=2)
    def _pair(s):
        in_cp(s + 1, 1).start()
        in_cp(s, 0).wait()

        @pl.when(s >= 2)
        def _():
            out_cp(s - 2, 0).wait()

        transpose(0)
        out_cp(s, 0).start()

        @pl.when(s + 2 < _SLABS_PER_W)
        def _():
            in_cp(s + 2, 0).start()

        in_cp(s + 1, 1).wait()

        @pl.when(s >= 2)
        def _():
            out_cp(s - 1, 1).wait()

        transpose(1)
        out_cp(s + 1, 1).start()

    out_cp(_SLABS_PER_W - 2, 0).wait()
    out_cp(_SLABS_PER_W - 1, 1).wait()


def _gather_body(fwd_hbm, tbl_hbm, out_hbm, idx_hbm, raw_v, g_v, rows, sem_i,
                 sem_g, sem_s):
    wid = lax.axis_index("s") * _NC + lax.axis_index("c")
    t0 = wid * _OSLABS_PER_W
    lanes = lax.iota(jnp.int32, _L)

    pltpu.sync_copy(fwd_hbm.at[pl.ds(t0, _OSLABS_PER_W)], raw_v)
    idx_cp = pltpu.async_copy(raw_v, idx_hbm.at[pl.ds(t0, _OSLABS_PER_W)],
                              sem_i)
    for j in range(_OSLABS_PER_W):
        for i in range(_B // _L):
            g_v[j, pl.ds(i * _L, _L)] = (
                raw_v[j, pl.ds(i * _L, _L)] * _B + lanes + i * _L
            )

    def start_gather(j):
        return pltpu.async_copy(tbl_hbm.at[g_v.at[j]], rows[j % _NBUF],
                                sem_g[j % _NBUF])

    gathers = {j: start_gather(j) for j in range(_NBUF)}
    stores = {}
    for j in range(_OSLABS_PER_W):
        if j >= 1 and j + _NBUF - 1 < _OSLABS_PER_W:
            stores[j - 1].wait()
            gathers[j + _NBUF - 1] = start_gather(j + _NBUF - 1)
        gathers[j].wait()
        stores[j] = pltpu.async_copy(
            rows[j % _NBUF],
            out_hbm.at[pl.ds((t0 + j) * _B, _B)],
            sem_s[j % _NBUF])
    for j in range(_OSLABS_PER_W - min(_NBUF, _OSLABS_PER_W), _OSLABS_PER_W):
        stores[j].wait()
    idx_cp.wait()


@jax.jit
def _shuffle(fwd, src):
    mesh = plsc.VectorSubcoreMesh(core_axis_name="c", subcore_axis_name="s")
    table = pl.kernel(
        _tr_body,
        out_type=jax.ShapeDtypeStruct((_T * _B, _C), jnp.float32),
        mesh=mesh,
        compiler_params=pltpu.CompilerParams(use_tc_tiling_on_sc=False,
                                             needs_layout_passes=False),
        scratch_types=[
            [pltpu.VMEM((_C, _SP), jnp.float32) for _ in range(2)],
            [pltpu.VMEM((_B, _C), jnp.float32) for _ in range(2)],
            [pltpu.SemaphoreType.DMA for _ in range(2)],
            [pltpu.SemaphoreType.DMA for _ in range(2)],
        ],
    )(src)
    out, idx = pl.kernel(
        _gather_body,
        out_type=(
            jax.ShapeDtypeStruct((_REMAIN * _B, _C), jnp.float32),
            jax.ShapeDtypeStruct((_REMAIN, _B), jnp.int32),
        ),
        mesh=mesh,
        compiler_params=pltpu.CompilerParams(use_tc_tiling_on_sc=False),
        scratch_types=[
            pltpu.VMEM((_OSLABS_PER_W, _B), jnp.int32),
            pltpu.VMEM((_OSLABS_PER_W, _B), jnp.int32),
            [pltpu.VMEM((_B, _C), jnp.float32) for _ in range(_NBUF)],
            pltpu.SemaphoreType.DMA,
            [pltpu.SemaphoreType.DMA for _ in range(_NBUF)],
            [pltpu.SemaphoreType.DMA for _ in range(_NBUF)],
        ],
    )(fwd, table)
    return out, idx


def kernel(patches, forward_indexes):
    src = jnp.swapaxes(patches, 1, 2).reshape(_T * _C, _B)
    fwd = forward_indexes.reshape(_T, _B)
    out, idx = _shuffle(fwd, src)
    return out.reshape(_REMAIN, _B, _C), idx


# speedup vs baseline: 1.4397x; 1.4397x over previous
import jax
import jax.numpy as jnp
from jax import lax
from jax.experimental import pallas as pl
from jax.experimental.pallas import tpu as pltpu
from jax.experimental.pallas import tpu_sc as plsc

_T, _B, _C = 1024, 128, 192
_REMAIN = _T - (_T * 3) // 4
_ROWS = _REMAIN * _B
_NC, _NS = 2, 16
_NW = _NC * _NS
_SLABS_PER_W = _T // _NW
_L = 16
_SP = _B + 1
_RING = 144
_FCH = 64


def _body(fwd_hbm, src_hbm, out_hbm, idx_hbm, fch_v, pos_v, in_bufs, dense_v,
          sb_v, so_v, oidx_v, oidx2_v, sem_i, sem_in, sem_sc):
    wid = lax.axis_index("s") * _NC + lax.axis_index("c")
    s0 = wid * _SLABS_PER_W
    lanes = lax.iota(jnp.int32, _L)
    rowm = [lanes + _L * m for m in range(_C // _L)]

    idx_cp = pltpu.async_copy(fwd_hbm.at[pl.ds(wid * 8, 8)],
                              idx_hbm.at[pl.ds(wid * 8, 8)], sem_i)

    for r in range(_SLABS_PER_W):
        for i in range(_B // _L):
            pos_v[r, pl.ds(i * _L, _L)] = jnp.full((_L,), -1, jnp.int32)

    for cc in range(_REMAIN // _FCH):
        pltpu.sync_copy(fwd_hbm.at[pl.ds(cc * _FCH, _FCH)], fch_v)

        @pl.loop(0, _FCH)
        def _row(r):
            tvec = jnp.full((_L,), cc * _FCH, jnp.int32) + r
            for i in range(_B // _L):
                val = fch_v[r, pl.ds(i * _L, _L)]
                loc = val - s0
                m = (loc >= 0) & (loc < _SLABS_PER_W)
                plsc.store_scatter(pos_v, [loc, lanes + i * _L], tvec, mask=m)

    def in_cp(j, k):
        return pltpu.make_async_copy(
            src_hbm.at[pl.ds((s0 + j) * _C, _C)],
            in_bufs[k].at[:, pl.ds(0, _B)], sem_in[k])

    def scatter_cp():
        return pltpu.make_async_copy(
            dense_v.at[pl.ds(0, 128)], out_hbm.at[oidx2_v.at[0]], sem_sc)

    def flush():
        for mm in range(128 // _L):
            oidx2_v[0, pl.ds(mm * _L, _L)] = oidx_v[pl.ds(mm * _L, _L)]
        scatter_cp().start()
        scatter_cp().wait()
        @pl.loop(0, _L)
        def _mv(rr):
            for m2 in range(_C // _L):
                dense_v[rr, pl.ds(m2 * _L, _L)] = \
                    dense_v[128 + rr, pl.ds(m2 * _L, _L)]
        oidx_v[pl.ds(0, _L)] = oidx_v[pl.ds(128, _L)]

    def do_slab(j, k, p):
        in_cp(j, k).wait()
        q = jnp.int32(0)
        for i in range(_B // _L):
            posrow = pos_v[j, pl.ds(i * _L, _L)]
            m = posrow >= 0
            bvec = lanes + i * _L
            plsc.store_compressed(sb_v.at[pl.ds(q, _L)], bvec, mask=m)
            plsc.store_compressed(so_v.at[pl.ds(q, _L)],
                                  posrow * _B + bvec, mask=m)
            q = q + plsc.all_reduce_population_count(m)[0]

        ngroups = (q + _L - 1) // _L

        @pl.loop(0, ngroups, init_carry=p)
        def _grp(g, pc):
            cnt = jnp.minimum(q - g * _L, _L)
            ovec = so_v[pl.ds(g * _L, _L)]
            sbvec = sb_v[pl.ds(g * _L, _L)]
            for k16 in range(_L):
                @pl.when(g * _L + k16 < q)
                def _():
                    b_s = sbvec[k16]
                    col = jnp.full((_L,), b_s, jnp.int32)
                    slot = pc + k16
                    for m in range(_C // _L):
                        v = plsc.load_gather(in_bufs[k], [rowm[m], col])
                        dense_v[slot, pl.ds(m * _L, _L)] = v
            oidx_v[pl.ds(pc, _L)] = ovec
            pa = pc + cnt

            @pl.when(pa >= 128)
            def _():
                flush()

            return jnp.where(pa >= 128, pa - 128, pa)

        return _grp

    in_cp(0, 0).start()

    @pl.loop(0, _SLABS_PER_W, step=2, init_carry=jnp.int32(0))
    def _pair(s, p):
        in_cp(s + 1, 1).start()
        p = do_slab(s, 0, p)

        @pl.when(s + 2 < _SLABS_PER_W)
        def _():
            in_cp(s + 2, 0).start()

        return do_slab(s + 1, 1, p)

    p = _pair

    for mm in range(128 // _L):
        iv = oidx_v[pl.ds(mm * _L, _L)]
        slot_id = lanes + mm * _L
        oidx2_v[0, pl.ds(mm * _L, _L)] = jnp.where(slot_id < p, iv, _ROWS)
    scatter_cp().start()
    scatter_cp().wait()
    idx_cp.wait()


@jax.jit
def _shuffle(fwd, src):
    mesh = plsc.VectorSubcoreMesh(core_axis_name="c", subcore_axis_name="s")
    out, idx = pl.kernel(
        _body,
        out_type=(
            jax.ShapeDtypeStruct((_ROWS + 1, _C), jnp.float32),
            jax.ShapeDtypeStruct((_REMAIN, _B), jnp.int32),
        ),
        mesh=mesh,
        compiler_params=pltpu.CompilerParams(use_tc_tiling_on_sc=False,
                                             needs_layout_passes=False),
        scratch_types=[
            pltpu.VMEM((_FCH, _B), jnp.int32),
            pltpu.VMEM((_SLABS_PER_W, _B), jnp.int32),
            [pltpu.VMEM((_C, _SP), jnp.float32) for _ in range(2)],
            pltpu.VMEM((_RING, _C), jnp.float32),
            pltpu.VMEM((_RING,), jnp.int32),
            pltpu.VMEM((_RING,), jnp.int32),
            pltpu.VMEM((_RING,), jnp.int32),
            pltpu.VMEM((1, 128), jnp.int32),
            pltpu.SemaphoreType.DMA,
            [pltpu.SemaphoreType.DMA for _ in range(2)],
            pltpu.SemaphoreType.DMA,
        ],
    )(fwd, src)
    return out, idx


def kernel(patches, forward_indexes):
    src = jnp.swapaxes(patches, 1, 2).reshape(_T * _C, _B)
    out, idx = _shuffle(forward_indexes, src)
    return out[:_ROWS].reshape(_REMAIN, _B, _C), idx


# speedup vs baseline: 1.8248x; 1.2674x over previous
import jax
import jax.numpy as jnp
from jax import lax
from jax.experimental import pallas as pl
from jax.experimental.pallas import tpu as pltpu
from jax.experimental.pallas import tpu_sc as plsc

_T, _B, _C = 1024, 128, 192
_REMAIN = _T - (_T * 3) // 4
_ROWS = _REMAIN * _B
_NC, _NS = 2, 16
_NW = _NC * _NS
_SLABS_PER_W = _T // _NW
_L = 16
_SP = _B + 1
_RING = 144
_FCH = 64


def _body(fwd_hbm, src_hbm, out_hbm, idx_hbm, fch_v, pos_v, in_bufs, dense_v,
          sb_v, so_v, oidx_v, oidx2_v, sem_i, sem_in, sem_sc):
    wid = lax.axis_index("s") * _NC + lax.axis_index("c")
    s0 = wid * _SLABS_PER_W
    lanes = lax.iota(jnp.int32, _L)
    rowm = [lanes + _L * m for m in range(_C // _L)]

    idx_cp = pltpu.async_copy(fwd_hbm.at[pl.ds(wid * 8, 8)],
                              idx_hbm.at[pl.ds(wid * 8, 8)], sem_i)

    for r in range(_SLABS_PER_W):
        for i in range(_B // _L):
            pos_v[r, pl.ds(i * _L, _L)] = jnp.full((_L,), -1, jnp.int32)
    for i in range(_RING // _L):
        sb_v[pl.ds(i * _L, _L)] = jnp.zeros((_L,), jnp.int32)

    for cc in range(_REMAIN // _FCH):
        pltpu.sync_copy(fwd_hbm.at[pl.ds(cc * _FCH, _FCH)], fch_v)

        @pl.loop(0, _FCH)
        def _row(r):
            tvec = jnp.full((_L,), cc * _FCH, jnp.int32) + r
            for i in range(_B // _L):
                val = fch_v[r, pl.ds(i * _L, _L)]
                loc = val - s0
                m = (loc >= 0) & (loc < _SLABS_PER_W)
                plsc.store_scatter(pos_v, [loc, lanes + i * _L], tvec, mask=m)

    def in_cp(j, k):
        return pltpu.make_async_copy(
            src_hbm.at[pl.ds((s0 + j) * _C, _C)],
            in_bufs[k].at[:, pl.ds(0, _B)], sem_in[k])

    def scatter_cp():
        return pltpu.make_async_copy(
            dense_v.at[pl.ds(0, 128)], out_hbm.at[oidx2_v.at[0]], sem_sc)

    def flush():
        for mm in range(128 // _L):
            oidx2_v[0, pl.ds(mm * _L, _L)] = oidx_v[pl.ds(mm * _L, _L)]
        scatter_cp().start()
        scatter_cp().wait()
        @pl.loop(0, _L)
        def _mv(rr):
            for m2 in range(_C // _L):
                dense_v[rr, pl.ds(m2 * _L, _L)] = \
                    dense_v[128 + rr, pl.ds(m2 * _L, _L)]
        oidx_v[pl.ds(0, _L)] = oidx_v[pl.ds(128, _L)]

    def do_slab(j, k, p):
        in_cp(j, k).wait()
        q = jnp.int32(0)
        for i in range(_B // _L):
            posrow = pos_v[j, pl.ds(i * _L, _L)]
            m = posrow >= 0
            bvec = lanes + i * _L
            plsc.store_compressed(sb_v.at[pl.ds(q, _L)], bvec, mask=m)
            plsc.store_compressed(so_v.at[pl.ds(q, _L)],
                                  posrow * _B + bvec, mask=m)
            q = q + plsc.all_reduce_population_count(m)[0]

        ngroups = (q + _L - 1) // _L

        @pl.loop(0, ngroups, init_carry=p)
        def _grp(g, pc):
            cnt = jnp.minimum(q - g * _L, _L)
            ovec = so_v[pl.ds(g * _L, _L)]
            sbvec = sb_v[pl.ds(g * _L, _L)]
            for k16 in range(_L):
                b_s = sbvec[k16]
                col = jnp.full((_L,), b_s, jnp.int32)
                slot = pc + k16
                for m in range(_C // _L):
                    v = plsc.load_gather(in_bufs[k], [rowm[m], col])
                    dense_v[slot, pl.ds(m * _L, _L)] = v
            oidx_v[pl.ds(pc, _L)] = ovec
            pa = pc + cnt

            @pl.when(pa >= 128)
            def _():
                flush()

            return jnp.where(pa >= 128, pa - 128, pa)

        return _grp

    in_cp(0, 0).start()

    @pl.loop(0, _SLABS_PER_W, step=2, init_carry=jnp.int32(0))
    def _pair(s, p):
        in_cp(s + 1, 1).start()
        p = do_slab(s, 0, p)

        @pl.when(s + 2 < _SLABS_PER_W)
        def _():
            in_cp(s + 2, 0).start()

        return do_slab(s + 1, 1, p)

    p = _pair

    @pl.when(p > 0)
    def _():
        lastvec = oidx_v[pl.ds(p - 1, _L)]
        o_dup = lastvec[0]

        @pl.loop(0, 128)
        def _dup(rr):
            @pl.when(rr >= p)
            def _():
                for m2 in range(_C // _L):
                    dense_v[rr, pl.ds(m2 * _L, _L)] = \
                        dense_v[p - 1, pl.ds(m2 * _L, _L)]

        for mm in range(128 // _L):
            iv = oidx_v[pl.ds(mm * _L, _L)]
            slot_id = lanes + mm * _L
            oidx2_v[0, pl.ds(mm * _L, _L)] = jnp.where(slot_id < p, iv, o_dup)
        scatter_cp().start()
        scatter_cp().wait()
    idx_cp.wait()


@jax.jit
def _shuffle(fwd, src):
    mesh = plsc.VectorSubcoreMesh(core_axis_name="c", subcore_axis_name="s")
    out, idx = pl.kernel(
        _body,
        out_type=(
            jax.ShapeDtypeStruct((_ROWS, _C), jnp.float32),
            jax.ShapeDtypeStruct((_REMAIN, _B), jnp.int32),
        ),
        mesh=mesh,
        compiler_params=pltpu.CompilerParams(use_tc_tiling_on_sc=False,
                                             needs_layout_passes=False),
        scratch_types=[
            pltpu.VMEM((_FCH, _B), jnp.int32),
            pltpu.VMEM((_SLABS_PER_W, _B), jnp.int32),
            [pltpu.VMEM((_C, _SP), jnp.float32) for _ in range(2)],
            pltpu.VMEM((_RING, _C), jnp.float32),
            pltpu.VMEM((_RING,), jnp.int32),
            pltpu.VMEM((_RING,), jnp.int32),
            pltpu.VMEM((_RING,), jnp.int32),
            pltpu.VMEM((1, 128), jnp.int32),
            pltpu.SemaphoreType.DMA,
            [pltpu.SemaphoreType.DMA for _ in range(2)],
            pltpu.SemaphoreType.DMA,
        ],
    )(fwd, src)
    return out, idx


def kernel(patches, forward_indexes):
    src = jnp.swapaxes(patches, 1, 2).reshape(_T * _C, _B)
    out, idx = _shuffle(forward_indexes, src)
    return out.reshape(_REMAIN, _B, _C), idx
